# baseline (speedup 1.0000x reference)
import functools

import jax
import jax.numpy as jnp
from jax import lax
from jax.experimental import pallas as pl
from jax.experimental.pallas import tpu as pltpu
from jax.experimental.pallas import tpu_sc as plsc

N = 8192
E = 16384
G = 256
D = 64
ED = 12
EH = 12
HID = 256
STEPS = 3
TS = 6

W = 128
NW = 32
BPW = 2 * E // NW
CH = BPW // 128
RBLK = 1024
MBLK = 2048
NBLK = N // RBLK
NRB = NBLK


@functools.cache
def _sc_kernels():
    mesh = plsc.VectorSubcoreMesh(core_axis_name="c", subcore_axis_name="s")

    @functools.partial(
        pl.kernel,
        out_type=jax.ShapeDtypeStruct((2 * E, W), jnp.float32),
        mesh=mesh,
        scratch_types=[
            pltpu.VMEM((CH, 128), jnp.int32),
            pltpu.VMEM((512, W), jnp.float32),
            pltpu.SemaphoreType.DMA,
        ],
        compiler_params=pltpu.CompilerParams(use_tc_tiling_on_sc=True),
    )
    def sc_gather(table_hbm, idx_hbm, out_hbm, idx_v, rows_v, sem):
        wid = lax.axis_index("c") * 16 + lax.axis_index("s")
        pltpu.sync_copy(idx_hbm.at[wid], idx_v)
        for half in range(2):
            cps = []
            for j in range(CH // 2):
                cps.append(pltpu.async_copy(
                    table_hbm.at[idx_v.at[half * (CH // 2) + j]],
                    rows_v.at[pl.ds(j * 128, 128)], sem))
            for c in cps:
                c.wait()
            pltpu.sync_copy(
                rows_v, out_hbm.at[pl.ds(wid * BPW + half * 512, 512)])

    @functools.partial(
        pl.kernel,
        out_type=jax.ShapeDtypeStruct((2 * N, D), jnp.float32),
        mesh=mesh,
        scratch_types=[
            pltpu.VMEM((CH, 128), jnp.int32),
            pltpu.VMEM((512, D), jnp.float32),
            pltpu.VMEM_SHARED((N, D), jnp.float32),
        ],
        compiler_params=pltpu.CompilerParams(use_tc_tiling_on_sc=False),
    )
    def sc_scatter(msg_hbm, idx_hbm, zero_hbm, out_hbm, idx_v, msg_v, acc_sh):
        c = lax.axis_index("c")
        s = lax.axis_index("s")
        wid = c * 16 + s
        for q in range(4):
            pltpu.sync_copy(zero_hbm,
                            acc_sh.at[pl.ds(s * 512 + q * 128, 128)])
        plsc.subcore_barrier()
        pltpu.sync_copy(idx_hbm.at[wid], idx_v)
        for half in range(2):
            pltpu.sync_copy(
                msg_hbm.at[pl.ds(wid * BPW + half * 512, 512), pl.ds(0, D)],
                msg_v)
            for j in range(CH // 2):
                pltpu.sync_copy(msg_v.at[pl.ds(j * 128, 128)],
                                acc_sh.at[idx_v.at[half * (CH // 2) + j]],
                                add=True)
        plsc.subcore_barrier()
        pltpu.sync_copy(acc_sh.at[pl.ds(s * 512, 512)],
                        out_hbm.at[pl.ds(c * N + s * 512, 512)])

    return sc_gather, sc_scatter


def _dot(a, b):
    return lax.dot_general(a, b, (((1,), (0,)), ((), ())),
                           preferred_element_type=jnp.float32)


def _dot_t(a, b):
    return lax.dot_general(a, b, (((1,), (1,)), ((), ())),
                           preferred_element_type=jnp.float32)


def _dot_lt(a, b):
    return lax.dot_general(a, b, (((0,), (0,)), ((), ())),
                           preferred_element_type=jnp.float32)


def _proj_body(x_ref, w_ref, b_ref, o_ref):
    o_ref[...] = jax.nn.relu(_dot_lt(x_ref[...], w_ref[0]) + b_ref[0, 0])


def _proj(xst, w_s, b_s):
    nb = 2 * N // RBLK
    return pl.pallas_call(
        _proj_body,
        grid=(2, nb // 2),
        in_specs=[
            pl.BlockSpec((D, RBLK), lambda b, i: (0, b * (nb // 2) + i)),
            pl.BlockSpec((1, D, W), lambda b, i: (b, 0, 0)),
            pl.BlockSpec((1, 1, W), lambda b, i: (b, 0, 0)),
        ],
        out_specs=pl.BlockSpec((RBLK, W), lambda b, i: (b * (nb // 2) + i, 0)),
        out_shape=jax.ShapeDtypeStruct((2 * N, W), jnp.float32),
    )(xst, w_s, b_s)


def _msg_body(hs_ref, ea_ref, w1_ref, b1_ref, t2_ref, bm_ref, r_ref, s_ref,
              o_ref):
    hs = hs_ref[:, 0:D]
    zed = jax.nn.relu(_dot_lt(ea_ref[...], w1_ref[0]) + b1_ref[0, 0])
    p = _dot(hs, t2_ref[0])
    z = _dot(zed, r_ref[...])
    o_ref[:, 0:D] = _dot(z * p, s_ref[...]) + _dot(hs, bm_ref[0])
    o_ref[:, D:W] = jnp.zeros((MBLK, W - D), jnp.float32)


def _msg(hs, ea_s, w1_s, b1_s, t2_s, bm_s, rmat, smat):
    eb = 2 * E // MBLK
    return pl.pallas_call(
        _msg_body,
        grid=(2, eb // 2),
        in_specs=[
            pl.BlockSpec((MBLK, W), lambda b, i: (b * (eb // 2) + i, 0)),
            pl.BlockSpec((ED, MBLK), lambda b, i: (0, b * (eb // 2) + i)),
            pl.BlockSpec((1, ED, EH), lambda b, i: (b, 0, 0)),
            pl.BlockSpec((1, 1, EH), lambda b, i: (b, 0, 0)),
            pl.BlockSpec((1, D, EH * D), lambda b, i: (b, 0, 0)),
            pl.BlockSpec((1, D, D), lambda b, i: (b, 0, 0)),
            pl.BlockSpec((EH, EH * D), lambda b, i: (0, 0)),
            pl.BlockSpec((EH * D, D), lambda b, i: (0, 0)),
        ],
        out_specs=pl.BlockSpec((MBLK, W), lambda b, i: (b * (eb // 2) + i, 0)),
        out_shape=jax.ShapeDtypeStruct((2 * E, W), jnp.float32),
    )(hs, ea_s, w1_s, b1_s, t2_s, bm_s, rmat, smat)


def _gru_math(x, h, wi, wh, bi, bh):
    r = jax.nn.sigmoid(_dot_t(x, wi[0:D]) + bi[:, 0:D]
                       + _dot_t(h, wh[0:D]) + bh[:, 0:D])
    z = jax.nn.sigmoid(_dot_t(x, wi[D:2 * D]) + bi[:, D:2 * D]
                       + _dot_t(h, wh[D:2 * D]) + bh[:, D:2 * D])
    n = jnp.tanh(_dot_t(x, wi[2 * D:]) + bi[:, 2 * D:]
                 + r * (_dot_t(h, wh[2 * D:]) + bh[:, 2 * D:]))
    return (1.0 - z) * n + z * h


def _gru_body(a_ref, h_ref, bc_ref, wi_ref, wh_ref, bi_ref, bh_ref, o_ref):
    m = jax.nn.relu(a_ref[...] + bc_ref[0, 0])
    h = h_ref[:, 0:D]
    out = _gru_math(m, h, wi_ref[0], wh_ref[0], bi_ref[0], bh_ref[0])
    o_ref[:, 0:D] = out
    o_ref[:, D:W] = jnp.zeros((RBLK, W - D), jnp.float32)


def _gru_step(agg, hid, bc_s, wi_s, wh_s, bi_s, bh_s):
    nb = 2 * N // RBLK
    return pl.pallas_call(
        _gru_body,
        grid=(2, nb // 2),
        in_specs=[
            pl.BlockSpec((RBLK, D), lambda b, i: (b * (nb // 2) + i, 0)),
            pl.BlockSpec((RBLK, W), lambda b, i: (b * (nb // 2) + i, 0)),
            pl.BlockSpec((1, 1, D), lambda b, i: (b, 0, 0)),
            pl.BlockSpec((1, 3 * D, D), lambda b, i: (b, 0, 0)),
            pl.BlockSpec((1, 3 * D, D), lambda b, i: (b, 0, 0)),
            pl.BlockSpec((1, 1, 3 * D), lambda b, i: (b, 0, 0)),
            pl.BlockSpec((1, 1, 3 * D), lambda b, i: (b, 0, 0)),
        ],
        out_specs=pl.BlockSpec((RBLK, W), lambda b, i: (b * (nb // 2) + i, 0)),
        out_shape=jax.ShapeDtypeStruct((2 * N, W), jnp.float32),
    )(agg, hid, bc_s, wi_s, wh_s, bi_s, bh_s)


def _readout_body(nf_ref, ng_ref, *refs):
    wrefs = refs[:8 * TS]
    g_ref = refs[8 * TS]
    ms_sc, ss_sc, gf_sc, gr_sc = refs[8 * TS + 1:]

    def oht_at(i):
        row = ng_ref[pl.ds(i, 1), :]
        io = lax.broadcasted_iota(jnp.int32, (G, RBLK), 0).astype(jnp.float32)
        return jnp.where(io == row, 1.0, 0.0)

    gf_sc[...] = jnp.zeros((G, D), jnp.float32)
    def init_blk(i, _):
        gf_sc[...] += _dot(oht_at(i), nf_ref[pl.ds(i * RBLK, RBLK), 0:D])
        return 0
    lax.fori_loop(0, NRB, init_blk, 0)

    for ts in range(TS):
        (wl_ref, bl_ref, wn_ref, bn_ref,
         wi_ref, wh_ref, bi_ref, bh_ref) = wrefs[8 * ts:8 * ts + 8]
        gf = gf_sc[...]
        gv = _dot_t(wl_ref[:, 0:D], jax.nn.relu(gf))
        blv = bl_ref[0, 0]

        ms_sc[...] = jnp.full((G, 1), -1e30, jnp.float32)
        ss_sc[...] = jnp.zeros((G, 1), jnp.float32)
        gr_sc[...] = jnp.zeros((G, D), jnp.float32)
        def p1(i, _):
            oht = oht_at(i)
            nf = nf_ref[pl.ds(i * RBLK, RBLK), 0:D]
            lg = _dot(gv, oht) + _dot_t(wl_ref[:, D:2 * D], nf) + blv
            lg = jnp.where(lg >= 0, lg, 0.01 * lg)
            masked = jnp.where(oht > 0.5, lg, -1e30)
            m_old = ms_sc[...]
            m_new = jnp.maximum(m_old,
                                jnp.max(masked, axis=1, keepdims=True))
            scale = jnp.exp(m_old - m_new)
            mb = jnp.sum(oht * m_new, axis=0, keepdims=True)
            ex = jnp.exp(lg - mb)
            exm = oht * ex
            hv = _dot(nf, wn_ref[...]) + bn_ref[...]
            ms_sc[...] = m_new
            ss_sc[...] = ss_sc[...] * scale + jnp.sum(exm, axis=1,
                                                      keepdims=True)
            gr_sc[...] = gr_sc[...] * scale + _dot(exm, hv)
            return 0
        lax.fori_loop(0, NRB, p1, 0)
        ssv = ss_sc[...]
        grep = jnp.where(ssv > 0, gr_sc[...] / ssv, 0.0)
        grep = jnp.where(grep > 0, grep, jnp.exp(jnp.minimum(grep, 0.0)) - 1.0)

        gf_sc[...] = _gru_math(grep, gf, wi_ref, wh_ref,
                               bi_ref[...], bh_ref[...])

    g_ref[...] = gf_sc[...]


def _readout(node, branch, ng8, wts):
    nspec = [
        pl.BlockSpec((N, W), lambda g: (branch, 0)),
        pl.BlockSpec((NRB, RBLK), lambda g: (0, 0)),
    ] + [pl.BlockSpec(w.shape, lambda g, nd=w.ndim: (0,) * nd) for w in wts]
    return pl.pallas_call(
        _readout_body,
        grid=(1,),
        in_specs=nspec,
        out_specs=pl.BlockSpec((G, D), lambda g: (0, 0)),
        out_shape=jax.ShapeDtypeStruct((G, D), jnp.float32),
        scratch_shapes=[
            pltpu.VMEM((G, 1), jnp.float32),
            pltpu.VMEM((G, 1), jnp.float32),
            pltpu.VMEM((G, D), jnp.float32),
            pltpu.VMEM((G, D), jnp.float32),
        ],
    )(node, ng8, *wts)


def _pred_body(g1_ref, g2_ref, mc_ref, w1a_ref, w1b_ref, w1c_ref, b1_ref,
               gm_ref, bt_ref, w2_ref, b2_ref, o_ref):
    h = jax.nn.relu(_dot(g1_ref[...], w1a_ref[...])
                    + _dot(g2_ref[...], w1b_ref[...])
                    + _dot(mc_ref[...], w1c_ref[...]) + b1_ref[...])
    h = h * (gm_ref[...] / jnp.sqrt(1.0 + 1e-5)) + bt_ref[...]
    o_ref[...] = _dot(h, w2_ref[...]) + b2_ref[...]


def _pred(g1, g2, mc, w1a, w1b, w1c, b1, gm, bt, w2, b2):
    return pl.pallas_call(
        _pred_body,
        out_shape=jax.ShapeDtypeStruct((G, 2), jnp.float32),
    )(g1, g2, mc, w1a, w1b, w1c, b1, gm, bt, w2, b2)


def kernel(x1, edge_index1, edge_attr1, node2graph1,
           x2, edge_index2, edge_attr2, node2graph2, mc, params):
    m1, m2 = params['mpnn1'], params['mpnn2']
    f32 = jnp.float32

    xst = jnp.concatenate([x1.T, x2.T], axis=1)
    eat = jnp.concatenate([edge_attr1.T, edge_attr2.T], axis=1)
    src3 = jnp.concatenate(
        [edge_index1[0], edge_index2[0] + N]).reshape(NW, CH, 128)
    dst3 = jnp.concatenate(
        [edge_index1[1], edge_index2[1]]).reshape(NW, CH, 128)
    zeros128 = jnp.zeros((128, D), f32)

    def padw(a):
        return jnp.pad(a, ((0, 0), (0, W - D)))

    wp_s = jnp.stack([padw(m1['Wp']), padw(m2['Wp'])])
    bp_s = jnp.stack([jnp.pad(m1['bp'], (0, W - D)),
                      jnp.pad(m2['bp'], (0, W - D))]).reshape(2, 1, W)
    w1_s = jnp.stack([m1['W1'], m2['W1']])
    b1_s = jnp.stack([m1['b1'], m2['b1']]).reshape(2, 1, EH)

    def t2(m):
        return m['W2'].reshape(EH, D, D).transpose(1, 0, 2).reshape(D, EH * D)

    t2_s = jnp.stack([t2(m1), t2(m2)])
    bm_s = jnp.stack([m1['b2'].reshape(D, D), m2['b2'].reshape(D, D)])
    rmat = jnp.repeat(jnp.eye(EH, dtype=f32), D, axis=1)
    smat = jnp.tile(jnp.eye(D, dtype=f32), (EH, 1))
    bc_s = jnp.stack([m1['b_conv'], m2['b_conv']]).reshape(2, 1, D)
    wi_s = jnp.stack([m1['gru']['W_ih'], m2['gru']['W_ih']])
    wh_s = jnp.stack([m1['gru']['W_hh'], m2['gru']['W_hh']])
    bi_s = jnp.stack([m1['gru']['b_ih'], m2['gru']['b_ih']]).reshape(2, 1, 3 * D)
    bh_s = jnp.stack([m1['gru']['b_hh'], m2['gru']['b_hh']]).reshape(2, 1, 3 * D)

    def ro_wts(plist):
        wts = []
        for p in plist:
            wts += [p['Wl'].T, p['bl'].reshape(1, 1), p['Wn'],
                    p['bn'].reshape(1, D),
                    p['gru']['W_ih'], p['gru']['W_hh'],
                    p['gru']['b_ih'].reshape(1, 3 * D),
                    p['gru']['b_hh'].reshape(1, 3 * D)]
        return wts

    sc_gather, sc_scatter = _sc_kernels()
    node = _proj(xst, wp_s, bp_s)
    hidden = node
    for _ in range(STEPS):
        hs = sc_gather(node, src3)
        msg = _msg(hs, eat, w1_s, b1_s, t2_s, bm_s, rmat, smat)
        agg = sc_scatter(msg, dst3, zeros128)
        node = _gru_step(agg, hidden, bc_s, wi_s, wh_s, bi_s, bh_s)
        hidden = node

    ng81 = node2graph1.astype(f32).reshape(NRB, RBLK)
    ng82 = node2graph2.astype(f32).reshape(NRB, RBLK)
    r1 = _readout(node, 0, ng81, ro_wts(params['ro1']))
    r2 = _readout(node, 1, ng82, ro_wts(params['ro2']))

    pp = params['pred']
    return _pred(r1, r2, mc,
                 pp['W1'][0:D], pp['W1'][D:2 * D], pp['W1'][2 * D:],
                 pp['b1'].reshape(1, HID), pp['gamma'].reshape(1, HID),
                 pp['beta'].reshape(1, HID), pp['W2'], pp['b2'].reshape(1, 2))


# speedup vs baseline: 1.0746x; 1.0746x over previous
import functools

import jax
import jax.numpy as jnp
from jax import lax
from jax.experimental import pallas as pl
from jax.experimental.pallas import tpu as pltpu
from jax.experimental.pallas import tpu_sc as plsc

N = 8192
E = 16384
G = 256
D = 64
ED = 12
EH = 12
HID = 256
STEPS = 3
TS = 6

W = 128
NW = 32
BPW = 2 * E // NW
CH = BPW // 128
RBLK = 1024
MBLK = 2048
NBLK = N // RBLK
NRB = NBLK


@functools.cache
def _sc_kernels():
    mesh = plsc.VectorSubcoreMesh(core_axis_name="c", subcore_axis_name="s")

    @functools.partial(
        pl.kernel,
        out_type=jax.ShapeDtypeStruct((2 * E, W), jnp.float32),
        mesh=mesh,
        scratch_types=[
            pltpu.VMEM((CH, 128), jnp.int32),
            pltpu.VMEM((512, W), jnp.float32),
            pltpu.SemaphoreType.DMA,
        ],
        compiler_params=pltpu.CompilerParams(use_tc_tiling_on_sc=True),
    )
    def sc_gather(table_hbm, idx_hbm, out_hbm, idx_v, rows_v, sem):
        wid = lax.axis_index("c") * 16 + lax.axis_index("s")
        pltpu.sync_copy(idx_hbm.at[wid], idx_v)
        for half in range(2):
            cps = []
            for j in range(CH // 2):
                cps.append(pltpu.async_copy(
                    table_hbm.at[idx_v.at[half * (CH // 2) + j]],
                    rows_v.at[pl.ds(j * 128, 128)], sem))
            for c in cps:
                c.wait()
            pltpu.sync_copy(
                rows_v, out_hbm.at[pl.ds(wid * BPW + half * 512, 512)])

    @functools.partial(
        pl.kernel,
        out_type=jax.ShapeDtypeStruct((2 * N, W), jnp.float32),
        mesh=mesh,
        scratch_types=[
            pltpu.VMEM((CH, 128), jnp.int32),
            pltpu.VMEM((512, D), jnp.float32),
            pltpu.VMEM((512, D), jnp.float32),
            pltpu.VMEM_SHARED((N, D), jnp.float32),
        ],
        compiler_params=pltpu.CompilerParams(use_tc_tiling_on_sc=False),
    )
    def sc_scatter(msg_hbm, idx_hbm, zero_hbm, out_hbm,
                   idx_v, msg_v, z_v, acc_sh):
        c = lax.axis_index("c")
        s = lax.axis_index("s")
        wid = c * 16 + s
        for q in range(4):
            pltpu.sync_copy(zero_hbm,
                            acc_sh.at[pl.ds(s * 512 + q * 128, 128)])
            pltpu.sync_copy(zero_hbm, z_v.at[pl.ds(q * 128, 128)])
        plsc.subcore_barrier()
        pltpu.sync_copy(idx_hbm.at[wid], idx_v)
        for half in range(2):
            pltpu.sync_copy(
                msg_hbm.at[pl.ds(wid * BPW + half * 512, 512), pl.ds(0, D)],
                msg_v)
            for j in range(CH // 2):
                pltpu.sync_copy(msg_v.at[pl.ds(j * 128, 128)],
                                acc_sh.at[idx_v.at[half * (CH // 2) + j]],
                                add=True)
        plsc.subcore_barrier()
        rows = pl.ds(c * N + s * 512, 512)
        pltpu.sync_copy(acc_sh.at[pl.ds(s * 512, 512)],
                        out_hbm.at[rows, pl.ds(0, D)])
        pltpu.sync_copy(z_v, out_hbm.at[rows, pl.ds(D, D)])

    return sc_gather, sc_scatter


def _dot(a, b):
    return lax.dot_general(a, b, (((1,), (0,)), ((), ())),
                           preferred_element_type=jnp.float32)


def _dot_t(a, b):
    return lax.dot_general(a, b, (((1,), (1,)), ((), ())),
                           preferred_element_type=jnp.float32)


def _dot_lt(a, b):
    return lax.dot_general(a, b, (((0,), (0,)), ((), ())),
                           preferred_element_type=jnp.float32)


def _proj_body(x_ref, w_ref, b_ref, o_ref):
    o_ref[...] = jax.nn.relu(_dot_lt(x_ref[...], w_ref[0]) + b_ref[0, 0])


def _proj(xst, w_s, b_s):
    nb = 2 * N // RBLK
    return pl.pallas_call(
        _proj_body,
        grid=(2, nb // 2),
        in_specs=[
            pl.BlockSpec((D, RBLK), lambda b, i: (0, b * (nb // 2) + i)),
            pl.BlockSpec((1, D, W), lambda b, i: (b, 0, 0)),
            pl.BlockSpec((1, 1, W), lambda b, i: (b, 0, 0)),
        ],
        out_specs=pl.BlockSpec((RBLK, W), lambda b, i: (b * (nb // 2) + i, 0)),
        out_shape=jax.ShapeDtypeStruct((2 * N, W), jnp.float32),
    )(xst, w_s, b_s)


def _msg_body(hs_ref, ea_ref, w1_ref, b1_ref, t2_ref, bm_ref, r_ref, s_ref,
              o_ref):
    hs = hs_ref[:, 0:D]
    zed = jax.nn.relu(_dot_lt(ea_ref[...], w1_ref[0]) + b1_ref[0, 0])
    p = _dot(hs, t2_ref[0])
    z = _dot(zed, r_ref[...])
    o_ref[:, 0:D] = _dot(z * p, s_ref[...]) + _dot(hs, bm_ref[0])
    o_ref[:, D:W] = jnp.zeros((MBLK, W - D), jnp.float32)


def _msg(hs, ea_s, w1_s, b1_s, t2_s, bm_s, rmat, smat):
    eb = 2 * E // MBLK
    return pl.pallas_call(
        _msg_body,
        grid=(2, eb // 2),
        in_specs=[
            pl.BlockSpec((MBLK, W), lambda b, i: (b * (eb // 2) + i, 0)),
            pl.BlockSpec((ED, MBLK), lambda b, i: (0, b * (eb // 2) + i)),
            pl.BlockSpec((1, ED, EH), lambda b, i: (b, 0, 0)),
            pl.BlockSpec((1, 1, EH), lambda b, i: (b, 0, 0)),
            pl.BlockSpec((1, D, EH * D), lambda b, i: (b, 0, 0)),
            pl.BlockSpec((1, D, D), lambda b, i: (b, 0, 0)),
            pl.BlockSpec((EH, EH * D), lambda b, i: (0, 0)),
            pl.BlockSpec((EH * D, D), lambda b, i: (0, 0)),
        ],
        out_specs=pl.BlockSpec((MBLK, W), lambda b, i: (b * (eb // 2) + i, 0)),
        out_shape=jax.ShapeDtypeStruct((2 * E, W), jnp.float32),
    )(hs, ea_s, w1_s, b1_s, t2_s, bm_s, rmat, smat)


def _gru_math(x, h, wi, wh, bi, bh):
    r = jax.nn.sigmoid(_dot_t(x, wi[0:D]) + bi[:, 0:D]
                       + _dot_t(h, wh[0:D]) + bh[:, 0:D])
    z = jax.nn.sigmoid(_dot_t(x, wi[D:2 * D]) + bi[:, D:2 * D]
                       + _dot_t(h, wh[D:2 * D]) + bh[:, D:2 * D])
    n = jnp.tanh(_dot_t(x, wi[2 * D:]) + bi[:, 2 * D:]
                 + r * (_dot_t(h, wh[2 * D:]) + bh[:, 2 * D:]))
    return (1.0 - z) * n + z * h


def _gru_body(a_ref, h_ref, bc_ref, wi_ref, wh_ref, bi_ref, bh_ref, o_ref):
    m = jax.nn.relu(a_ref[:, 0:D] + bc_ref[0, 0])
    h = h_ref[:, 0:D]
    out = _gru_math(m, h, wi_ref[0], wh_ref[0], bi_ref[0], bh_ref[0])
    o_ref[:, 0:D] = out
    o_ref[:, D:W] = jnp.zeros((RBLK, W - D), jnp.float32)


def _gru_step(agg, hid, bc_s, wi_s, wh_s, bi_s, bh_s):
    nb = 2 * N // RBLK
    return pl.pallas_call(
        _gru_body,
        grid=(2, nb // 2),
        in_specs=[
            pl.BlockSpec((RBLK, W), lambda b, i: (b * (nb // 2) + i, 0)),
            pl.BlockSpec((RBLK, W), lambda b, i: (b * (nb // 2) + i, 0)),
            pl.BlockSpec((1, 1, D), lambda b, i: (b, 0, 0)),
            pl.BlockSpec((1, 3 * D, D), lambda b, i: (b, 0, 0)),
            pl.BlockSpec((1, 3 * D, D), lambda b, i: (b, 0, 0)),
            pl.BlockSpec((1, 1, 3 * D), lambda b, i: (b, 0, 0)),
            pl.BlockSpec((1, 1, 3 * D), lambda b, i: (b, 0, 0)),
        ],
        out_specs=pl.BlockSpec((RBLK, W), lambda b, i: (b * (nb // 2) + i, 0)),
        out_shape=jax.ShapeDtypeStruct((2 * N, W), jnp.float32),
    )(agg, hid, bc_s, wi_s, wh_s, bi_s, bh_s)


def _readout_body(nf_ref, ng_ref, *refs):
    wrefs = refs[:8 * TS]
    g_ref = refs[8 * TS]
    ms_sc, ss_sc, gf_sc, gr_sc = refs[8 * TS + 1:]

    def oht_at(i):
        row = ng_ref[pl.ds(i, 1), :]
        io = lax.broadcasted_iota(jnp.int32, (G, RBLK), 0).astype(jnp.float32)
        return jnp.where(io == row, 1.0, 0.0)

    gf_sc[...] = jnp.zeros((G, D), jnp.float32)
    def init_blk(i, _):
        gf_sc[...] += _dot(oht_at(i), nf_ref[pl.ds(i * RBLK, RBLK), 0:D])
        return 0
    lax.fori_loop(0, NRB, init_blk, 0)

    for ts in range(TS):
        (wl_ref, bl_ref, wn_ref, bn_ref,
         wi_ref, wh_ref, bi_ref, bh_ref) = wrefs[8 * ts:8 * ts + 8]
        gf = gf_sc[...]
        gv = _dot_t(wl_ref[:, 0:D], jax.nn.relu(gf))
        blv = bl_ref[0, 0]

        ms_sc[...] = jnp.full((G, 1), -1e30, jnp.float32)
        ss_sc[...] = jnp.zeros((G, 1), jnp.float32)
        gr_sc[...] = jnp.zeros((G, D), jnp.float32)
        def p1(i, _):
            oht = oht_at(i)
            nf = nf_ref[pl.ds(i * RBLK, RBLK), 0:D]
            lg = _dot(gv, oht) + _dot_t(wl_ref[:, D:2 * D], nf) + blv
            lg = jnp.where(lg >= 0, lg, 0.01 * lg)
            masked = jnp.where(oht > 0.5, lg, -1e30)
            m_old = ms_sc[...]
            m_new = jnp.maximum(m_old,
                                jnp.max(masked, axis=1, keepdims=True))
            scale = jnp.exp(m_old - m_new)
            mb = jnp.sum(oht * m_new, axis=0, keepdims=True)
            ex = jnp.exp(lg - mb)
            exm = oht * ex
            hv = _dot(nf, wn_ref[...]) + bn_ref[...]
            ms_sc[...] = m_new
            ss_sc[...] = ss_sc[...] * scale + jnp.sum(exm, axis=1,
                                                      keepdims=True)
            gr_sc[...] = gr_sc[...] * scale + _dot(exm, hv)
            return 0
        lax.fori_loop(0, NRB, p1, 0)
        ssv = ss_sc[...]
        grep = jnp.where(ssv > 0, gr_sc[...] / ssv, 0.0)
        grep = jnp.where(grep > 0, grep, jnp.exp(jnp.minimum(grep, 0.0)) - 1.0)

        gf_sc[...] = _gru_math(grep, gf, wi_ref, wh_ref,
                               bi_ref[...], bh_ref[...])

    g_ref[...] = gf_sc[...]


def _readout(node, branch, ng8, wts):
    nspec = [
        pl.BlockSpec((N, W), lambda g: (branch, 0)),
        pl.BlockSpec((NRB, RBLK), lambda g: (0, 0)),
    ] + [pl.BlockSpec(w.shape, lambda g, nd=w.ndim: (0,) * nd) for w in wts]
    return pl.pallas_call(
        _readout_body,
        grid=(1,),
        in_specs=nspec,
        out_specs=pl.BlockSpec((G, D), lambda g: (0, 0)),
        out_shape=jax.ShapeDtypeStruct((G, D), jnp.float32),
        scratch_shapes=[
            pltpu.VMEM((G, 1), jnp.float32),
            pltpu.VMEM((G, 1), jnp.float32),
            pltpu.VMEM((G, D), jnp.float32),
            pltpu.VMEM((G, D), jnp.float32),
        ],
    )(node, ng8, *wts)


def _pred_body(g1_ref, g2_ref, mc_ref, w1a_ref, w1b_ref, w1c_ref, b1_ref,
               gm_ref, bt_ref, w2_ref, b2_ref, o_ref):
    h = jax.nn.relu(_dot(g1_ref[...], w1a_ref[...])
                    + _dot(g2_ref[...], w1b_ref[...])
                    + _dot(mc_ref[...], w1c_ref[...]) + b1_ref[...])
    h = h * (gm_ref[...] / jnp.sqrt(1.0 + 1e-5)) + bt_ref[...]
    o_ref[...] = _dot(h, w2_ref[...]) + b2_ref[...]


def _pred(g1, g2, mc, w1a, w1b, w1c, b1, gm, bt, w2, b2):
    return pl.pallas_call(
        _pred_body,
        out_shape=jax.ShapeDtypeStruct((G, 2), jnp.float32),
    )(g1, g2, mc, w1a, w1b, w1c, b1, gm, bt, w2, b2)


def kernel(x1, edge_index1, edge_attr1, node2graph1,
           x2, edge_index2, edge_attr2, node2graph2, mc, params):
    m1, m2 = params['mpnn1'], params['mpnn2']
    f32 = jnp.float32

    xst = jnp.concatenate([x1.T, x2.T], axis=1)
    eat = jnp.concatenate([edge_attr1.T, edge_attr2.T], axis=1)
    src3 = jnp.concatenate(
        [edge_index1[0], edge_index2[0] + N]).reshape(NW, CH, 128)
    dst3 = jnp.concatenate(
        [edge_index1[1], edge_index2[1]]).reshape(NW, CH, 128)
    zeros128 = jnp.zeros((128, D), f32)

    def padw(a):
        return jnp.pad(a, ((0, 0), (0, W - D)))

    wp_s = jnp.stack([padw(m1['Wp']), padw(m2['Wp'])])
    bp_s = jnp.stack([jnp.pad(m1['bp'], (0, W - D)),
                      jnp.pad(m2['bp'], (0, W - D))]).reshape(2, 1, W)
    w1_s = jnp.stack([m1['W1'], m2['W1']])
    b1_s = jnp.stack([m1['b1'], m2['b1']]).reshape(2, 1, EH)

    def t2(m):
        return m['W2'].reshape(EH, D, D).transpose(1, 0, 2).reshape(D, EH * D)

    t2_s = jnp.stack([t2(m1), t2(m2)])
    bm_s = jnp.stack([m1['b2'].reshape(D, D), m2['b2'].reshape(D, D)])
    rmat = jnp.repeat(jnp.eye(EH, dtype=f32), D, axis=1)
    smat = jnp.tile(jnp.eye(D, dtype=f32), (EH, 1))
    bc_s = jnp.stack([m1['b_conv'], m2['b_conv']]).reshape(2, 1, D)
    wi_s = jnp.stack([m1['gru']['W_ih'], m2['gru']['W_ih']])
    wh_s = jnp.stack([m1['gru']['W_hh'], m2['gru']['W_hh']])
    bi_s = jnp.stack([m1['gru']['b_ih'], m2['gru']['b_ih']]).reshape(2, 1, 3 * D)
    bh_s = jnp.stack([m1['gru']['b_hh'], m2['gru']['b_hh']]).reshape(2, 1, 3 * D)

    def ro_wts(plist):
        wts = []
        for p in plist:
            wts += [p['Wl'].T, p['bl'].reshape(1, 1), p['Wn'],
                    p['bn'].reshape(1, D),
                    p['gru']['W_ih'], p['gru']['W_hh'],
                    p['gru']['b_ih'].reshape(1, 3 * D),
                    p['gru']['b_hh'].reshape(1, 3 * D)]
        return wts

    sc_gather, sc_scatter = _sc_kernels()
    node = _proj(xst, wp_s, bp_s)
    hidden = node
    for _ in range(STEPS):
        hs = sc_gather(node, src3)
        msg = _msg(hs, eat, w1_s, b1_s, t2_s, bm_s, rmat, smat)
        agg = sc_scatter(msg, dst3, zeros128)
        node = _gru_step(agg, hidden, bc_s, wi_s, wh_s, bi_s, bh_s)
        hidden = node

    ng81 = node2graph1.astype(f32).reshape(NRB, RBLK)
    ng82 = node2graph2.astype(f32).reshape(NRB, RBLK)
    r1 = _readout(node, 0, ng81, ro_wts(params['ro1']))
    r2 = _readout(node, 1, ng82, ro_wts(params['ro2']))

    pp = params['pred']
    return _pred(r1, r2, mc,
                 pp['W1'][0:D], pp['W1'][D:2 * D], pp['W1'][2 * D:],
                 pp['b1'].reshape(1, HID), pp['gamma'].reshape(1, HID),
                 pp['beta'].reshape(1, HID), pp['W2'], pp['b2'].reshape(1, 2))
